# Initial kernel scaffold; baseline (speedup 1.0000x reference)
import jax
import jax.numpy as jnp
from jax.experimental import pallas as pl


def kernel(edge_index, user_emb, item_emb, W1, b1, W2, b2):
    raise NotImplementedError("write your pallas kernel here")


# speedup vs baseline: 7.9336x; 7.9336x over previous
import jax
import jax.numpy as jnp
from jax import lax
from jax.experimental import pallas as pl
from jax.experimental.pallas import tpu as pltpu
from jax.experimental.pallas import tpu_sc as plsc

N = 10000
D = 128
E = 320000

NC = 2
NS = 16
NW = NC * NS
C = 128
NCH = 80
ET = NCH * C
EP = NW * ET
NP = 10240
RPT = NP // NS
ZR = 32

_sc_mesh = plsc.VectorSubcoreMesh(core_axis_name="c", subcore_axis_name="s")


def _deg_body(dst_hbm, dp_hbm, dst_v, ones_v, zb_v, hist, ):
    cid = lax.axis_index("c")
    sid = lax.axis_index("s")
    wid = sid * NC + cid

    def fill_ones(i, _):
        ones_v[pl.ds(i * 16, 16)] = jnp.ones((16,), jnp.float32)
        return 0
    lax.fori_loop(0, C // 16, fill_ones, 0)

    def fill_zero(i, _):
        zb_v[pl.ds(i * 16, 16)] = jnp.zeros((16,), jnp.float32)
        return 0
    lax.fori_loop(0, RPT // 16, fill_zero, 0)
    pltpu.sync_copy(zb_v, hist.at[pl.ds(sid * RPT, RPT)])
    plsc.subcore_barrier()

    pltpu.sync_copy(dst_hbm.at[wid], dst_v)

    def chunk(j, _):
        pltpu.sync_copy(ones_v, hist.at[dst_v.at[j]], add=True)
        return 0
    lax.fori_loop(0, NCH, chunk, 0)

    plsc.subcore_barrier()
    pltpu.sync_copy(hist.at[pl.ds(sid * RPT, RPT)],
                    dp_hbm.at[cid, pl.ds(sid * RPT, RPT)])


_deg = pl.kernel(
    _deg_body,
    out_type=jax.ShapeDtypeStruct((NC, NP), jnp.float32),
    mesh=_sc_mesh,
    scratch_types=[
        pltpu.VMEM((NCH, C), jnp.int32),
        pltpu.VMEM((C,), jnp.float32),
        pltpu.VMEM((RPT,), jnp.float32),
        pltpu.VMEM_SHARED((NP,), jnp.float32),
    ],
)


def _agg_body(y_hbm, src_hbm, dst_hbm, p_hbm, src_v, dst_v, buf, zbuf, acc, sem):
    cid = lax.axis_index("c")
    sid = lax.axis_index("s")
    wid = sid * NC + cid

    def fill_zero(i, _):
        r = i // (D // 16)
        c0 = (i % (D // 16)) * 16
        zbuf[r, pl.ds(c0, 16)] = jnp.zeros((16,), jnp.float32)
        return 0
    lax.fori_loop(0, ZR * (D // 16), fill_zero, 0)
    for k in range(RPT // ZR):
        pltpu.sync_copy(zbuf, acc.at[pl.ds(sid * RPT + k * ZR, ZR)])
    plsc.subcore_barrier()

    pltpu.sync_copy(src_hbm.at[wid], src_v)
    pltpu.sync_copy(dst_hbm.at[wid], dst_v)

    def chunk(j, _):
        pltpu.async_copy(y_hbm.at[src_v.at[j]], buf, sem).wait()
        pltpu.sync_copy(buf, acc.at[dst_v.at[j]], add=True)
        return 0
    lax.fori_loop(0, NCH, chunk, 0)

    plsc.subcore_barrier()
    pltpu.sync_copy(acc.at[pl.ds(sid * RPT, RPT)],
                    p_hbm.at[cid, pl.ds(sid * RPT, RPT)])


_agg = pl.kernel(
    _agg_body,
    out_type=jax.ShapeDtypeStruct((NC, NP, D), jnp.float32),
    mesh=_sc_mesh,
    scratch_types=[
        pltpu.VMEM((NCH, C), jnp.int32),
        pltpu.VMEM((NCH, C), jnp.int32),
        pltpu.VMEM((C, D), jnp.float32),
        pltpu.VMEM((ZR, D), jnp.float32),
        pltpu.VMEM_SHARED((NP, D), jnp.float32),
        pltpu.SemaphoreType.DMA,
    ],
)

BR = 256
GR = NP // BR


def _dinv_of(dg):
    return lax.rsqrt(dg[:, 0:1] + dg[:, 1:2] + 1.0)


def _mm1_body(x_ref, w_ref, dg_ref, y_ref):
    dinv = _dinv_of(dg_ref[...])
    xw = jnp.dot(x_ref[...], w_ref[...], preferred_element_type=jnp.float32)
    y_ref[...] = xw * dinv


_mm1 = pl.pallas_call(
    _mm1_body,
    out_shape=jax.ShapeDtypeStruct((NP, D), jnp.float32),
    grid=(GR,),
    in_specs=[
        pl.BlockSpec((BR, D), lambda i: (i, 0)),
        pl.BlockSpec((D, D), lambda i: (0, 0)),
        pl.BlockSpec((BR, NC), lambda i: (i, 0)),
    ],
    out_specs=pl.BlockSpec((BR, D), lambda i: (i, 0)),
)


def _mm2_body(y_ref, p_ref, dg_ref, b_ref, w_ref, o_ref):
    dinv = _dinv_of(dg_ref[...])
    agg = y_ref[...] + p_ref[0] + p_ref[1]
    h = jnp.maximum(agg * dinv + b_ref[...], 0.0)
    o_ref[...] = jnp.dot(h, w_ref[...], preferred_element_type=jnp.float32) * dinv


_mm2 = pl.pallas_call(
    _mm2_body,
    out_shape=jax.ShapeDtypeStruct((NP, D), jnp.float32),
    grid=(GR,),
    in_specs=[
        pl.BlockSpec((BR, D), lambda i: (i, 0)),
        pl.BlockSpec((NC, BR, D), lambda i: (0, i, 0)),
        pl.BlockSpec((BR, NC), lambda i: (i, 0)),
        pl.BlockSpec((1, D), lambda i: (0, 0)),
        pl.BlockSpec((D, D), lambda i: (0, 0)),
    ],
    out_specs=pl.BlockSpec((BR, D), lambda i: (i, 0)),
)


def _mm3_body(y_ref, p_ref, dg_ref, b_ref, o_ref):
    dinv = _dinv_of(dg_ref[...])
    agg = y_ref[...] + p_ref[0] + p_ref[1]
    o_ref[...] = agg * dinv + b_ref[...]


_mm3 = pl.pallas_call(
    _mm3_body,
    out_shape=jax.ShapeDtypeStruct((NP, D), jnp.float32),
    grid=(GR,),
    in_specs=[
        pl.BlockSpec((BR, D), lambda i: (i, 0)),
        pl.BlockSpec((NC, BR, D), lambda i: (0, i, 0)),
        pl.BlockSpec((BR, NC), lambda i: (i, 0)),
        pl.BlockSpec((1, D), lambda i: (0, 0)),
    ],
    out_specs=pl.BlockSpec((BR, D), lambda i: (i, 0)),
)


def kernel(edge_index, user_emb, item_emb, W1, b1, W2, b2):
    x = jnp.concatenate([user_emb, item_emb], axis=0)
    x = jnp.pad(x, ((0, NP - N), (0, 0)))
    src = edge_index[0].astype(jnp.int32)
    dst = edge_index[1].astype(jnp.int32)
    pad = jnp.full((EP - E,), NP - 1, jnp.int32)
    src_r = jnp.concatenate([src, pad]).reshape(NW, NCH, C)
    dst_r = jnp.concatenate([dst, pad]).reshape(NW, NCH, C)

    deg_t = _deg(dst_r).T
    y1 = _mm1(x, W1, deg_t)
    p = _agg(y1, src_r, dst_r)
    y2 = _mm2(y1, p, deg_t, b1.reshape(1, D), W2)
    q = _agg(y2, src_r, dst_r)
    out = _mm3(y2, q, deg_t, b2.reshape(1, D))
    return out[:N]


# speedup vs baseline: 8.5788x; 1.0813x over previous
import jax
import jax.numpy as jnp
from jax import lax
from jax.experimental import pallas as pl
from jax.experimental.pallas import tpu as pltpu
from jax.experimental.pallas import tpu_sc as plsc

N = 10000
D = 128
E = 320000

NC = 2
NS = 16
NW = NC * NS
C = 128
NCH = 80
ET = NCH * C
EP = NW * ET
NP = 10240
RPT = NP // NS
ZR = 32

_sc_mesh = plsc.VectorSubcoreMesh(core_axis_name="c", subcore_axis_name="s")


def _deg_body(dst_hbm, dp_hbm, dst_v, ones_v, zb_v, hist, ):
    cid = lax.axis_index("c")
    sid = lax.axis_index("s")
    wid = sid * NC + cid

    def fill_ones(i, _):
        ones_v[pl.ds(i * 16, 16)] = jnp.ones((16,), jnp.float32)
        return 0
    lax.fori_loop(0, C // 16, fill_ones, 0)

    def fill_zero(i, _):
        zb_v[pl.ds(i * 16, 16)] = jnp.zeros((16,), jnp.float32)
        return 0
    lax.fori_loop(0, RPT // 16, fill_zero, 0)
    pltpu.sync_copy(zb_v, hist.at[pl.ds(sid * RPT, RPT)])
    plsc.subcore_barrier()

    pltpu.sync_copy(dst_hbm.at[wid], dst_v)

    def chunk(j, _):
        pltpu.sync_copy(ones_v, hist.at[dst_v.at[j]], add=True)
        return 0
    lax.fori_loop(0, NCH, chunk, 0)

    plsc.subcore_barrier()
    pltpu.sync_copy(hist.at[pl.ds(sid * RPT, RPT)],
                    dp_hbm.at[cid, pl.ds(sid * RPT, RPT)])


_deg = pl.kernel(
    _deg_body,
    out_type=jax.ShapeDtypeStruct((NC, NP), jnp.float32),
    mesh=_sc_mesh,
    scratch_types=[
        pltpu.VMEM((NCH, C), jnp.int32),
        pltpu.VMEM((C,), jnp.float32),
        pltpu.VMEM((RPT,), jnp.float32),
        pltpu.VMEM_SHARED((NP,), jnp.float32),
    ],
)


def _agg_body(y_hbm, src_hbm, dst_hbm, p_hbm, dst_v, sbuf, buf, zbuf, acc,
              si0, si1, sg0, sg1, ss0, ss1):
    cid = lax.axis_index("c")
    sid = lax.axis_index("s")
    wid = sid * NC + cid
    sem_i = (si0, si1)
    sem_g = (sg0, sg1)
    sem_s = (ss0, ss1)

    def fill_zero(i, _):
        r = i // (D // 16)
        c0 = (i % (D // 16)) * 16
        zbuf[r, pl.ds(c0, 16)] = jnp.zeros((16,), jnp.float32)
        return 0
    lax.fori_loop(0, ZR * (D // 16), fill_zero, 0)
    for k in range(RPT // ZR):
        pltpu.sync_copy(zbuf, acc.at[pl.ds(sid * RPT + k * ZR, ZR)])
    plsc.subcore_barrier()

    pltpu.sync_copy(dst_hbm.at[wid], dst_v)

    def idx_load(j, b):
        pltpu.async_copy(src_hbm.at[wid, j], sbuf.at[b], sem_i[b])

    def idx_wait(j, b):
        pltpu.make_async_copy(src_hbm.at[wid, j], sbuf.at[b], sem_i[b]).wait()

    def gather_start(b):
        pltpu.async_copy(y_hbm.at[sbuf.at[b]], buf.at[b], sem_g[b])

    def gather_wait(b):
        pltpu.make_async_copy(y_hbm.at[sbuf.at[b]], buf.at[b], sem_g[b]).wait()

    def scat_start(j, b):
        pltpu.async_copy(buf.at[b], acc.at[dst_v.at[j]], sem_s[b], add=True)

    def scat_wait(j, b):
        pltpu.make_async_copy(buf.at[b], acc.at[dst_v.at[j]], sem_s[b]).wait()

    def slot(j, b):
        gather_wait(b)
        idx_load(j + 2, b)
        scat_wait(j - 1, 1 - b)
        idx_wait(j + 1, 1 - b)
        gather_start(1 - b)
        scat_start(j, b)

    idx_load(0, 0)
    idx_load(1, 1)
    idx_wait(0, 0)
    gather_start(0)

    gather_wait(0)
    idx_load(2, 0)
    idx_wait(1, 1)
    gather_start(1)
    scat_start(0, 0)

    def pair(i, _):
        slot(2 * i + 1, 1)
        slot(2 * i + 2, 0)
        return 0
    lax.fori_loop(0, (NCH - 2) // 2, pair, 0)

    slot(NCH - 1, 1)
    scat_wait(NCH - 1, 1)
    gather_wait(0)
    idx_wait(NCH + 1, 1)

    plsc.subcore_barrier()
    pltpu.sync_copy(acc.at[pl.ds(sid * RPT, RPT)],
                    p_hbm.at[cid, pl.ds(sid * RPT, RPT)])


_agg = pl.kernel(
    _agg_body,
    out_type=jax.ShapeDtypeStruct((NC, NP, D), jnp.float32),
    mesh=_sc_mesh,
    scratch_types=[
        pltpu.VMEM((NCH, C), jnp.int32),
        pltpu.VMEM((2, C), jnp.int32),
        pltpu.VMEM((2, C, D), jnp.float32),
        pltpu.VMEM((ZR, D), jnp.float32),
        pltpu.VMEM_SHARED((NP, D), jnp.float32),
        pltpu.SemaphoreType.DMA,
        pltpu.SemaphoreType.DMA,
        pltpu.SemaphoreType.DMA,
        pltpu.SemaphoreType.DMA,
        pltpu.SemaphoreType.DMA,
        pltpu.SemaphoreType.DMA,
    ],
)

BR = 256
GR = NP // BR


def _dinv_of(dg):
    return lax.rsqrt(dg[:, 0:1] + dg[:, 1:2] + 1.0)


def _mm1_body(x_ref, w_ref, dg_ref, y_ref):
    dinv = _dinv_of(dg_ref[...])
    xw = jnp.dot(x_ref[...], w_ref[...], preferred_element_type=jnp.float32)
    y_ref[...] = xw * dinv


_mm1 = pl.pallas_call(
    _mm1_body,
    out_shape=jax.ShapeDtypeStruct((NP, D), jnp.float32),
    grid=(GR,),
    in_specs=[
        pl.BlockSpec((BR, D), lambda i: (i, 0)),
        pl.BlockSpec((D, D), lambda i: (0, 0)),
        pl.BlockSpec((BR, NC), lambda i: (i, 0)),
    ],
    out_specs=pl.BlockSpec((BR, D), lambda i: (i, 0)),
)


def _mm2_body(y_ref, p_ref, dg_ref, b_ref, w_ref, o_ref):
    dinv = _dinv_of(dg_ref[...])
    agg = y_ref[...] + p_ref[0] + p_ref[1]
    h = jnp.maximum(agg * dinv + b_ref[...], 0.0)
    o_ref[...] = jnp.dot(h, w_ref[...], preferred_element_type=jnp.float32) * dinv


_mm2 = pl.pallas_call(
    _mm2_body,
    out_shape=jax.ShapeDtypeStruct((NP, D), jnp.float32),
    grid=(GR,),
    in_specs=[
        pl.BlockSpec((BR, D), lambda i: (i, 0)),
        pl.BlockSpec((NC, BR, D), lambda i: (0, i, 0)),
        pl.BlockSpec((BR, NC), lambda i: (i, 0)),
        pl.BlockSpec((1, D), lambda i: (0, 0)),
        pl.BlockSpec((D, D), lambda i: (0, 0)),
    ],
    out_specs=pl.BlockSpec((BR, D), lambda i: (i, 0)),
)


def _mm3_body(y_ref, p_ref, dg_ref, b_ref, o_ref):
    dinv = _dinv_of(dg_ref[...])
    agg = y_ref[...] + p_ref[0] + p_ref[1]
    o_ref[...] = agg * dinv + b_ref[...]


_mm3 = pl.pallas_call(
    _mm3_body,
    out_shape=jax.ShapeDtypeStruct((NP, D), jnp.float32),
    grid=(GR,),
    in_specs=[
        pl.BlockSpec((BR, D), lambda i: (i, 0)),
        pl.BlockSpec((NC, BR, D), lambda i: (0, i, 0)),
        pl.BlockSpec((BR, NC), lambda i: (i, 0)),
        pl.BlockSpec((1, D), lambda i: (0, 0)),
    ],
    out_specs=pl.BlockSpec((BR, D), lambda i: (i, 0)),
)


def kernel(edge_index, user_emb, item_emb, W1, b1, W2, b2):
    x = jnp.concatenate([user_emb, item_emb], axis=0)
    x = jnp.pad(x, ((0, NP - N), (0, 0)))
    src = edge_index[0].astype(jnp.int32)
    dst = edge_index[1].astype(jnp.int32)
    pad = jnp.full((EP - E,), NP - 1, jnp.int32)
    src_r = jnp.concatenate([src, pad]).reshape(NW, NCH, C)
    src_r = jnp.pad(src_r, ((0, 0), (0, 2), (0, 0)))
    dst_r = jnp.concatenate([dst, pad]).reshape(NW, NCH, C)

    deg_t = _deg(dst_r).T
    y1 = _mm1(x, W1, deg_t)
    p = _agg(y1, src_r, dst_r)
    y2 = _mm2(y1, p, deg_t, b1.reshape(1, D), W2)
    q = _agg(y2, src_r, dst_r)
    out = _mm3(y2, q, deg_t, b2.reshape(1, D))
    return out[:N]
